# baseline (speedup 1.0000x reference)
import jax
import jax.numpy as jnp
from jax.experimental import pallas as pl

_B = 256
_COLS = 3200
_NCHUNK = _COLS // 128


def _body(off_ref, x_ref, o_ref):
    off = off_ref[...]
    for c in range(_NCHUNK):
        sl = slice(c * 128, (c + 1) * 128)
        o_ref[:, sl] = x_ref[:, sl] - off


def kernel(x, W):
    del W
    n, dim, d = x.shape
    x2 = x.reshape(n, dim * d)
    off64 = x2[:, :d]
    off = jnp.concatenate([off64, off64], axis=1)

    grid = (n // _B, 2)
    out2 = pl.pallas_call(
        _body,
        grid=grid,
        in_specs=[
            pl.BlockSpec((_B, 128), lambda i, j: (i, 0)),
            pl.BlockSpec((_B, _COLS), lambda i, j: (i, j + 1)),
        ],
        out_specs=pl.BlockSpec((_B, _COLS), lambda i, j: (i, j)),
        out_shape=jax.ShapeDtypeStruct((n, 2 * _COLS), x.dtype),
    )(off, x2)
    return out2.reshape(n, 100, d)
